# initial kernel scaffold (unmeasured)
import jax
import jax.numpy as jnp
from jax import lax
from jax.experimental import pallas as pl
from jax.experimental.pallas import tpu as pltpu


def kernel(
    x,
):
    def body(*refs):
        pass

    out_shape = jax.ShapeDtypeStruct(..., jnp.float32)
    return pl.pallas_call(body, out_shape=out_shape)(...)


# baseline (device time: 32523 ns/iter reference)
import jax
import jax.numpy as jnp
from jax import lax
from jax.experimental import pallas as pl
from jax.experimental.pallas import tpu as pltpu


def kernel(x):
    _, m, n = x.shape

    def body(x_ref, out_ref, xsend, xrecv, ysend, yrecv,
             x_send_sem, x_recv_sem, y_send_sem, y_recv_sem):
        my_x = lax.axis_index("x")
        my_y = lax.axis_index("y")

        barrier = pltpu.get_barrier_semaphore()
        pl.semaphore_signal(barrier, inc=1, device_id=(1 - my_x, my_y),
                            device_id_type=pl.DeviceIdType.MESH)
        pl.semaphore_signal(barrier, inc=1, device_id=(my_x, 1 - my_y),
                            device_id_type=pl.DeviceIdType.MESH)
        pl.semaphore_wait(barrier, 2)

        xsend[...] = x_ref[0].astype(jnp.bfloat16)

        x_rdma = pltpu.make_async_remote_copy(
            src_ref=xsend, dst_ref=xrecv,
            send_sem=x_send_sem, recv_sem=x_recv_sem,
            device_id=(1 - my_x, my_y), device_id_type=pl.DeviceIdType.MESH)
        x_rdma.start()
        x_rdma.wait()

        partial = xsend[...] + xrecv[...]
        ysend[...] = partial
        out_ref[:, pl.ds(my_y * n, n)] = partial.astype(jnp.float32)

        y_rdma = pltpu.make_async_remote_copy(
            src_ref=ysend, dst_ref=yrecv,
            send_sem=y_send_sem, recv_sem=y_recv_sem,
            device_id=(my_x, 1 - my_y), device_id_type=pl.DeviceIdType.MESH)
        y_rdma.start()
        y_rdma.wait()

        out_ref[:, pl.ds((1 - my_y) * n, n)] = yrecv[...].astype(jnp.float32)

    return pl.pallas_call(
        body,
        out_shape=jax.ShapeDtypeStruct((m, 2 * n), jnp.float32),
        in_specs=[pl.BlockSpec(memory_space=pltpu.VMEM)],
        out_specs=pl.BlockSpec(memory_space=pltpu.VMEM),
        scratch_shapes=[
            pltpu.VMEM((m, n), jnp.bfloat16),
            pltpu.VMEM((m, n), jnp.bfloat16),
            pltpu.VMEM((m, n), jnp.bfloat16),
            pltpu.VMEM((m, n), jnp.bfloat16),
            pltpu.SemaphoreType.DMA,
            pltpu.SemaphoreType.DMA,
            pltpu.SemaphoreType.DMA,
            pltpu.SemaphoreType.DMA,
        ],
        compiler_params=pltpu.CompilerParams(collective_id=0),
    )(x)


# device time: 23870 ns/iter; 1.3625x vs baseline; 1.3625x over previous
import jax
import jax.numpy as jnp
from jax import lax
from jax.experimental import pallas as pl
from jax.experimental.pallas import tpu as pltpu

N_CHUNKS = 4


def kernel(x):
    _, m, n = x.shape
    mc = m // N_CHUNKS

    def body(x_ref, out_ref, xsend, xrecv, ysend, yrecv,
             x_send_sems, x_recv_sems, y_send_sems, y_recv_sems):
        my_x = lax.axis_index("x")
        my_y = lax.axis_index("y")
        x_partner = (1 - my_x, my_y)
        y_partner = (my_x, 1 - my_y)

        barrier = pltpu.get_barrier_semaphore()
        for nbr in (x_partner, y_partner):
            pl.semaphore_signal(barrier, inc=1, device_id=nbr,
                                device_id_type=pl.DeviceIdType.MESH)
        pl.semaphore_wait(barrier, 2)

        xsend[...] = x_ref[0].astype(jnp.bfloat16)

        def rows(c):
            return pl.ds(c * mc, mc)

        x_rdmas = []
        for c in range(N_CHUNKS):
            rdma = pltpu.make_async_remote_copy(
                src_ref=xsend.at[rows(c)], dst_ref=xrecv.at[rows(c)],
                send_sem=x_send_sems.at[c], recv_sem=x_recv_sems.at[c],
                device_id=x_partner, device_id_type=pl.DeviceIdType.MESH)
            rdma.start()
            x_rdmas.append(rdma)

        y_rdmas = []
        for c in range(N_CHUNKS):
            x_rdmas[c].wait_recv()
            partial = xsend[rows(c)] + xrecv[rows(c)]
            ysend[rows(c)] = partial
            rdma = pltpu.make_async_remote_copy(
                src_ref=ysend.at[rows(c)], dst_ref=yrecv.at[rows(c)],
                send_sem=y_send_sems.at[c], recv_sem=y_recv_sems.at[c],
                device_id=y_partner, device_id_type=pl.DeviceIdType.MESH)
            rdma.start()
            y_rdmas.append(rdma)
            out_ref[rows(c), pl.ds(my_y * n, n)] = partial.astype(jnp.float32)

        for c in range(N_CHUNKS):
            y_rdmas[c].wait_recv()
            out_ref[rows(c), pl.ds((1 - my_y) * n, n)] = (
                yrecv[rows(c)].astype(jnp.float32))

        for c in range(N_CHUNKS):
            x_rdmas[c].wait_send()
            y_rdmas[c].wait_send()

    return pl.pallas_call(
        body,
        out_shape=jax.ShapeDtypeStruct((m, 2 * n), jnp.float32),
        in_specs=[pl.BlockSpec(memory_space=pltpu.VMEM)],
        out_specs=pl.BlockSpec(memory_space=pltpu.VMEM),
        scratch_shapes=[
            pltpu.VMEM((m, n), jnp.bfloat16),
            pltpu.VMEM((m, n), jnp.bfloat16),
            pltpu.VMEM((m, n), jnp.bfloat16),
            pltpu.VMEM((m, n), jnp.bfloat16),
            pltpu.SemaphoreType.DMA((N_CHUNKS,)),
            pltpu.SemaphoreType.DMA((N_CHUNKS,)),
            pltpu.SemaphoreType.DMA((N_CHUNKS,)),
            pltpu.SemaphoreType.DMA((N_CHUNKS,)),
        ],
        compiler_params=pltpu.CompilerParams(collective_id=0),
    )(x)


# device time: 22513 ns/iter; 1.4446x vs baseline; 1.0603x over previous
import jax
import jax.numpy as jnp
from jax import lax
from jax.experimental import pallas as pl
from jax.experimental.pallas import tpu as pltpu

N_CHUNKS = 8


def kernel(x):
    _, m, n = x.shape
    mc = m // N_CHUNKS

    def body(x_ref, out_ref, xsend, xrecv, ysend, yrecv,
             x_send_sems, x_recv_sems, y_send_sems, y_recv_sems):
        my_x = lax.axis_index("x")
        my_y = lax.axis_index("y")
        x_partner = (1 - my_x, my_y)
        y_partner = (my_x, 1 - my_y)

        barrier = pltpu.get_barrier_semaphore()
        for nbr in (x_partner, y_partner):
            pl.semaphore_signal(barrier, inc=1, device_id=nbr,
                                device_id_type=pl.DeviceIdType.MESH)
        pl.semaphore_wait(barrier, 2)

        def rows(c):
            return pl.ds(c * mc, mc)

        x_rdmas = []
        for c in range(N_CHUNKS):
            xsend[rows(c)] = x_ref[0, rows(c), :].astype(jnp.bfloat16)
            rdma = pltpu.make_async_remote_copy(
                src_ref=xsend.at[rows(c)], dst_ref=xrecv.at[rows(c)],
                send_sem=x_send_sems.at[c], recv_sem=x_recv_sems.at[c],
                device_id=x_partner, device_id_type=pl.DeviceIdType.MESH)
            rdma.start()
            x_rdmas.append(rdma)

        y_rdmas = []
        for c in range(N_CHUNKS):
            x_rdmas[c].wait_recv()
            ysend[rows(c)] = xsend[rows(c)] + xrecv[rows(c)]
            rdma = pltpu.make_async_remote_copy(
                src_ref=ysend.at[rows(c)], dst_ref=yrecv.at[rows(c)],
                send_sem=y_send_sems.at[c], recv_sem=y_recv_sems.at[c],
                device_id=y_partner, device_id_type=pl.DeviceIdType.MESH)
            rdma.start()
            y_rdmas.append(rdma)

        for c in range(N_CHUNKS):
            out_ref[rows(c), pl.ds(my_y * n, n)] = (
                ysend[rows(c)].astype(jnp.float32))

        for c in range(N_CHUNKS):
            y_rdmas[c].wait_recv()
            out_ref[rows(c), pl.ds((1 - my_y) * n, n)] = (
                yrecv[rows(c)].astype(jnp.float32))

        for c in range(N_CHUNKS):
            x_rdmas[c].wait_send()
            y_rdmas[c].wait_send()

    return pl.pallas_call(
        body,
        out_shape=jax.ShapeDtypeStruct((m, 2 * n), jnp.float32),
        in_specs=[pl.BlockSpec(memory_space=pltpu.VMEM)],
        out_specs=pl.BlockSpec(memory_space=pltpu.VMEM),
        scratch_shapes=[
            pltpu.VMEM((m, n), jnp.bfloat16),
            pltpu.VMEM((m, n), jnp.bfloat16),
            pltpu.VMEM((m, n), jnp.bfloat16),
            pltpu.VMEM((m, n), jnp.bfloat16),
            pltpu.SemaphoreType.DMA((N_CHUNKS,)),
            pltpu.SemaphoreType.DMA((N_CHUNKS,)),
            pltpu.SemaphoreType.DMA((N_CHUNKS,)),
            pltpu.SemaphoreType.DMA((N_CHUNKS,)),
        ],
        compiler_params=pltpu.CompilerParams(collective_id=0),
    )(x)


# device time: 22500 ns/iter; 1.4455x vs baseline; 1.0006x over previous
import jax
import jax.numpy as jnp
from jax import lax
from jax.experimental import pallas as pl
from jax.experimental.pallas import tpu as pltpu

N_CHUNKS = 8


def kernel(x):
    _, m, n = x.shape
    mc = m // N_CHUNKS

    def body(x_ref, out_ref, xsend, xrecv, ysend, yrecv,
             x_send_sems, x_recv_sems, y_send_sems, y_recv_sems):
        my_x = lax.axis_index("x")
        my_y = lax.axis_index("y")
        x_partner = (1 - my_x, my_y)
        y_partner = (my_x, 1 - my_y)

        barrier = pltpu.get_barrier_semaphore()
        for nbr in (x_partner, y_partner):
            pl.semaphore_signal(barrier, inc=1, device_id=nbr,
                                device_id_type=pl.DeviceIdType.MESH)
        pl.semaphore_wait(barrier, 2)

        def rows(c):
            return pl.ds(c * mc, mc)

        x_rdmas = []
        for c in range(N_CHUNKS):
            xsend[rows(c)] = x_ref[0, rows(c), :].astype(jnp.bfloat16)
            rdma = pltpu.make_async_remote_copy(
                src_ref=xsend.at[rows(c)], dst_ref=xrecv.at[rows(c)],
                send_sem=x_send_sems.at[c], recv_sem=x_recv_sems.at[c],
                device_id=x_partner, device_id_type=pl.DeviceIdType.MESH)
            rdma.start()
            x_rdmas.append(rdma)

        y_rdmas = []
        for c in range(N_CHUNKS):
            x_rdmas[c].wait_recv()
            ysend[rows(c)] = xsend[rows(c)] + xrecv[rows(c)]
            rdma = pltpu.make_async_remote_copy(
                src_ref=ysend.at[rows(c)], dst_ref=yrecv.at[rows(c)],
                send_sem=y_send_sems.at[c], recv_sem=y_recv_sems.at[c],
                device_id=y_partner, device_id_type=pl.DeviceIdType.MESH)
            rdma.start()
            y_rdmas.append(rdma)

        for c in range(N_CHUNKS):
            pl.when(my_y == 0)(
                lambda c=c: out_ref.__setitem__(
                    (rows(c), slice(0, n)), ysend[rows(c)].astype(jnp.float32)))
            pl.when(my_y == 1)(
                lambda c=c: out_ref.__setitem__(
                    (rows(c), slice(n, 2 * n)), ysend[rows(c)].astype(jnp.float32)))

        for c in range(N_CHUNKS):
            y_rdmas[c].wait_recv()
            pl.when(my_y == 0)(
                lambda c=c: out_ref.__setitem__(
                    (rows(c), slice(n, 2 * n)), yrecv[rows(c)].astype(jnp.float32)))
            pl.when(my_y == 1)(
                lambda c=c: out_ref.__setitem__(
                    (rows(c), slice(0, n)), yrecv[rows(c)].astype(jnp.float32)))

        for c in range(N_CHUNKS):
            x_rdmas[c].wait_send()
            y_rdmas[c].wait_send()

    return pl.pallas_call(
        body,
        out_shape=jax.ShapeDtypeStruct((m, 2 * n), jnp.float32),
        in_specs=[pl.BlockSpec(memory_space=pltpu.VMEM)],
        out_specs=pl.BlockSpec(memory_space=pltpu.VMEM),
        scratch_shapes=[
            pltpu.VMEM((m, n), jnp.bfloat16),
            pltpu.VMEM((m, n), jnp.bfloat16),
            pltpu.VMEM((m, n), jnp.bfloat16),
            pltpu.VMEM((m, n), jnp.bfloat16),
            pltpu.SemaphoreType.DMA((N_CHUNKS,)),
            pltpu.SemaphoreType.DMA((N_CHUNKS,)),
            pltpu.SemaphoreType.DMA((N_CHUNKS,)),
            pltpu.SemaphoreType.DMA((N_CHUNKS,)),
        ],
        compiler_params=pltpu.CompilerParams(collective_id=0),
    )(x)
